# baseline (speedup 1.0000x reference)
import jax
import jax.numpy as jnp
from jax import lax
from jax.experimental import pallas as pl
from jax.experimental.pallas import tpu as pltpu
from jax.experimental.pallas import tpu_sc as plsc

N_NODES = 10000
N_EDGES = 320000
D_FEAT = 128
D_EDGE = 16
D_HID = 128
D_OUT = 128
D_PACK = D_HID // 2
D_ACC = D_HID

NC = 2
NS = 16
NW = NC * NS
PER_W = N_EDGES // NW
CH = 80
NCH = PER_W // CH
N_NODES_PAD = 10240
ROWS_PER_SUB = N_NODES_PAD // NS

def _node_proj_body(x_ref, w_ref, b_ref, u_ref):
    u = (
        jnp.dot(x_ref[...], w_ref[...], preferred_element_type=jnp.float32)
        + b_ref[...][None, :]
    )
    ub = u.astype(jnp.bfloat16).astype(jnp.float32)
    lo = lax.bitcast_convert_type(ub[:, :D_PACK], jnp.uint32)
    hi = lax.bitcast_convert_type(ub[:, D_PACK:], jnp.uint32)
    packed = (hi & jnp.uint32(0xFFFF0000)) | (lo >> jnp.uint32(16))
    u_ref[...] = lax.bitcast_convert_type(packed, jnp.float32)


def _node_proj(x, w1x, b1):
    blk = 1000
    return pl.pallas_call(
        _node_proj_body,
        grid=(N_NODES // blk,),
        in_specs=[
            pl.BlockSpec((blk, D_FEAT), lambda i: (i, 0)),
            pl.BlockSpec((D_FEAT, D_HID), lambda i: (0, 0)),
            pl.BlockSpec((D_HID,), lambda i: (0,)),
        ],
        out_specs=pl.BlockSpec((blk, D_PACK), lambda i: (i, 0)),
        out_shape=jax.ShapeDtypeStruct((N_NODES, D_PACK), jnp.float32),
    )(x, w1x, b1)


def _gather_body(u_hbm, col_hbm, g_hbm, idx_v, rows_v, sem):
    wid = lax.axis_index("s") * NC + lax.axis_index("c")
    base = wid * PER_W

    @pl.loop(0, NCH)
    def _(k):
        off = base + k * CH
        pltpu.sync_copy(col_hbm.at[pl.ds(off, CH)], idx_v)
        pltpu.async_copy(u_hbm.at[idx_v], rows_v, sem).wait()
        pltpu.sync_copy(rows_v, g_hbm.at[pl.ds(off, CH)])


def _gather(u, col):
    kfn = pl.kernel(
        _gather_body,
        out_type=jax.ShapeDtypeStruct((N_EDGES, D_PACK), jnp.float32),
        mesh=plsc.VectorSubcoreMesh(core_axis_name="c", subcore_axis_name="s"),
        scratch_types=[
            pltpu.VMEM((CH,), jnp.int32),
            pltpu.VMEM((CH, D_PACK), jnp.float32),
            pltpu.SemaphoreType.DMA,
        ],
        compiler_params=pltpu.CompilerParams(use_tc_tiling_on_sc=False),
    )
    return kfn(u, col)


def _edge_act_body(g_ref, ea_ref, w_ref, h_ref):
    packed = lax.bitcast_convert_type(g_ref[...], jnp.uint32)
    hi = lax.bitcast_convert_type(packed & jnp.uint32(0xFFFF0000), jnp.float32)
    lo = lax.bitcast_convert_type(packed << jnp.uint32(16), jnp.float32)
    g = jnp.concatenate([lo, hi], axis=1)
    t = jnp.dot(ea_ref[...], w_ref[...], preferred_element_type=jnp.float32)
    h_ref[...] = jax.nn.relu(g + t)


def _edge_act(g, ea, w1e):
    blk = 1000
    return pl.pallas_call(
        _edge_act_body,
        grid=(N_EDGES // blk,),
        in_specs=[
            pl.BlockSpec((blk, D_PACK), lambda i: (i, 0)),
            pl.BlockSpec((blk, D_EDGE), lambda i: (i, 0)),
            pl.BlockSpec((D_EDGE, D_HID), lambda i: (0, 0)),
        ],
        out_specs=pl.BlockSpec((blk, D_ACC), lambda i: (i, 0)),
        out_shape=jax.ShapeDtypeStruct((N_EDGES, D_ACC), jnp.float32),
    )(g, ea, w1e)


def _scatter_body(h_hbm, row_hbm, z_hbm, p_hbm, idx_v, buf_v, acc_sh):
    c = lax.axis_index("c")
    s = lax.axis_index("s")
    wid = s * NC + c

    pltpu.sync_copy(z_hbm, acc_sh.at[pl.ds(s * ROWS_PER_SUB, ROWS_PER_SUB)])
    plsc.subcore_barrier()

    base = wid * PER_W

    @pl.loop(0, NCH)
    def _(k):
        off = base + k * CH
        pltpu.sync_copy(row_hbm.at[pl.ds(off, CH)], idx_v)
        pltpu.sync_copy(h_hbm.at[pl.ds(off, CH)], buf_v)
        pltpu.sync_copy(buf_v, acc_sh.at[idx_v], add=True)

    plsc.subcore_barrier()
    pltpu.sync_copy(
        acc_sh.at[pl.ds(s * ROWS_PER_SUB, ROWS_PER_SUB)],
        p_hbm.at[c, pl.ds(s * ROWS_PER_SUB, ROWS_PER_SUB)],
    )


def _scatter(h, row, zeros_block):
    kfn = pl.kernel(
        _scatter_body,
        out_type=jax.ShapeDtypeStruct((NC, N_NODES_PAD, D_ACC), jnp.float32),
        mesh=plsc.VectorSubcoreMesh(core_axis_name="c", subcore_axis_name="s"),
        scratch_types=[
            pltpu.VMEM((CH,), jnp.int32),
            pltpu.VMEM((CH, D_ACC), jnp.float32),
            pltpu.VMEM_SHARED((N_NODES_PAD, D_ACC), jnp.float32),
        ],
    )
    return kfn(h, row, zeros_block)


def _combine_body(p_ref, w_ref, b_ref, o_ref):
    h = p_ref[0] + p_ref[1]
    o_ref[...] = jnp.dot(h, w_ref[...], preferred_element_type=jnp.float32)


def _combine(partials, w2, b2):
    blk = 1000
    return pl.pallas_call(
        _combine_body,
        grid=(N_NODES // blk,),
        in_specs=[
            pl.BlockSpec((NC, blk, D_ACC), lambda i: (0, i, 0)),
            pl.BlockSpec((D_HID, D_OUT), lambda i: (0, 0)),
            pl.BlockSpec((D_OUT,), lambda i: (0,)),
        ],
        out_specs=pl.BlockSpec((blk, D_OUT), lambda i: (i, 0)),
        out_shape=jax.ShapeDtypeStruct((N_NODES, D_OUT), jnp.float32),
    )(partials, w2, b2)


@jax.jit
def kernel(x, edge_index, edge_attr, W1, b1, W2, b2):
    row = edge_index[0].astype(jnp.int32)
    col = edge_index[1].astype(jnp.int32)
    w1x = W1[:D_FEAT, :]
    w1e = W1[D_FEAT:, :]
    zeros_block = jnp.zeros((ROWS_PER_SUB, D_ACC), jnp.float32)

    u = _node_proj(x, w1x, b1)
    g = _gather(u, col)
    h = _edge_act(g, edge_attr.astype(jnp.bfloat16), w1e.astype(jnp.bfloat16))
    partials = _scatter(h, row, zeros_block)
    return _combine(partials, W2, b2)


# speedup vs baseline: 1.5773x; 1.5773x over previous
import jax
import jax.numpy as jnp
from jax import lax
from jax.experimental import pallas as pl
from jax.experimental.pallas import tpu as pltpu
from jax.experimental.pallas import tpu_sc as plsc

N_NODES = 10000
N_EDGES = 320000
D_FEAT = 128
D_EDGE = 16
D_HID = 128
D_OUT = 128
D_ACC = D_HID

NC = 2
NS = 16
NW = NC * NS
PER_W = N_EDGES // NW
CH = 80
NCH = PER_W // CH
NBUF = 4
N_NODES_PAD = 10240
ROWS_PER_SUB = N_NODES_PAD // NS


def _node_proj_body(x_ref, w_ref, b_ref, u_ref):
    u_ref[...] = (
        jnp.dot(x_ref[...], w_ref[...], preferred_element_type=jnp.float32)
        + b_ref[...][None, :]
    )


def _node_proj(x, w1x, b1):
    blk = 1000
    return pl.pallas_call(
        _node_proj_body,
        grid=(N_NODES // blk,),
        in_specs=[
            pl.BlockSpec((blk, D_FEAT), lambda i: (i, 0)),
            pl.BlockSpec((D_FEAT, D_HID), lambda i: (0, 0)),
            pl.BlockSpec((D_HID,), lambda i: (0,)),
        ],
        out_specs=pl.BlockSpec((blk, D_HID), lambda i: (i, 0)),
        out_shape=jax.ShapeDtypeStruct((N_NODES, D_HID), jnp.float32),
    )(x, w1x, b1)


def _gather_body(u_hbm, col3_hbm, g_hbm, idxs_v, r0, r1, r2, r3, semg, semw):
    wid = lax.axis_index("s") * NC + lax.axis_index("c")
    base = wid * PER_W
    pltpu.sync_copy(col3_hbm.at[wid], idxs_v)

    bufs = (r0, r1, r2, r3)

    def start_gather(j, b):
        pltpu.async_copy(u_hbm.at[idxs_v.at[j]], bufs[b], semg)

    def wait_gather(b):
        pltpu.make_async_copy(u_hbm.at[pl.ds(0, CH)], bufs[b], semg).wait()

    def start_wb(j, b):
        pltpu.async_copy(bufs[b], g_hbm.at[pl.ds(base + j * CH, CH)], semw)

    def wait_wb(b):
        pltpu.make_async_copy(bufs[b], g_hbm.at[pl.ds(base, CH)], semw).wait()

    start_gather(0, 0)
    start_gather(1, 1)

    @pl.loop(0, (NCH - 1) // NBUF)
    def _(k):
        j0 = k * NBUF
        for t in range(NBUF):
            j = j0 + t
            b = t
            wait_gather(b)

            @pl.when(j >= 2)
            def _():
                wait_wb((t + 2) % NBUF)

            start_wb(j, b)

            @pl.when(j + 2 < NCH)
            def _():
                start_gather(j + 2, (t + 2) % NBUF)

    wait_gather(0)
    wait_wb(2)
    start_wb(NCH - 1, 0)
    wait_wb(3)
    wait_wb(0)


def _gather(u, col3):
    kfn = pl.kernel(
        _gather_body,
        out_type=jax.ShapeDtypeStruct((N_EDGES, D_HID), jnp.float32),
        mesh=plsc.VectorSubcoreMesh(core_axis_name="c", subcore_axis_name="s"),
        scratch_types=[
            pltpu.VMEM((NCH, CH), jnp.int32),
            pltpu.VMEM((CH, D_HID), jnp.float32),
            pltpu.VMEM((CH, D_HID), jnp.float32),
            pltpu.VMEM((CH, D_HID), jnp.float32),
            pltpu.VMEM((CH, D_HID), jnp.float32),
            pltpu.SemaphoreType.DMA,
            pltpu.SemaphoreType.DMA,
        ],
    )
    return kfn(u, col3)


def _edge_act_body(g_ref, ea_ref, w_ref, h_ref):
    t = jnp.dot(ea_ref[...], w_ref[...], preferred_element_type=jnp.float32)
    h_ref[...] = jax.nn.relu(g_ref[...] + t)


def _edge_act(g, ea, w1e):
    blk = 1000
    return pl.pallas_call(
        _edge_act_body,
        grid=(N_EDGES // blk,),
        in_specs=[
            pl.BlockSpec((blk, D_HID), lambda i: (i, 0)),
            pl.BlockSpec((blk, D_EDGE), lambda i: (i, 0)),
            pl.BlockSpec((D_EDGE, D_HID), lambda i: (0, 0)),
        ],
        out_specs=pl.BlockSpec((blk, D_ACC), lambda i: (i, 0)),
        out_shape=jax.ShapeDtypeStruct((N_EDGES, D_ACC), jnp.float32),
    )(g, ea, w1e)


def _scatter_body(h_hbm, row3_hbm, z_hbm, p_hbm, idxs_v, r0, r1, r2, acc_sh, seml, sems):
    c = lax.axis_index("c")
    s = lax.axis_index("s")
    wid = s * NC + c
    base = wid * PER_W

    pltpu.sync_copy(z_hbm, acc_sh.at[pl.ds(s * ROWS_PER_SUB, ROWS_PER_SUB)])
    pltpu.sync_copy(row3_hbm.at[wid], idxs_v)
    plsc.subcore_barrier()

    bufs = (r0, r1, r2)

    def start_load(j, b):
        pltpu.async_copy(h_hbm.at[pl.ds(base + j * CH, CH)], bufs[b], seml)

    def wait_load(b):
        pltpu.make_async_copy(h_hbm.at[pl.ds(base, CH)], bufs[b], seml).wait()

    def start_scat(j, b):
        pltpu.async_copy(bufs[b], acc_sh.at[idxs_v.at[j]], sems, add=True)

    def wait_scat(b):
        pltpu.make_async_copy(bufs[b], acc_sh.at[pl.ds(0, CH)], sems).wait()

    start_load(0, 0)
    start_load(1, 1)

    @pl.loop(0, 41)
    def _(k):
        j0 = k * 3
        for t in range(3):
            j = j0 + t
            b = t
            wait_load(b)

            @pl.when(j >= 1)
            def _():
                wait_scat((t + 2) % 3)

            start_scat(j, b)

            @pl.when(j + 2 < NCH)
            def _():
                start_load(j + 2, (t + 2) % 3)

    wait_load(0)
    wait_scat(2)
    start_scat(123, 0)
    wait_load(1)
    wait_scat(0)
    start_scat(124, 1)
    wait_scat(1)

    plsc.subcore_barrier()
    pltpu.sync_copy(
        acc_sh.at[pl.ds(s * ROWS_PER_SUB, ROWS_PER_SUB)],
        p_hbm.at[c, pl.ds(s * ROWS_PER_SUB, ROWS_PER_SUB)],
    )


def _scatter(h, row3, zeros_block):
    kfn = pl.kernel(
        _scatter_body,
        out_type=jax.ShapeDtypeStruct((NC, N_NODES_PAD, D_ACC), jnp.float32),
        mesh=plsc.VectorSubcoreMesh(core_axis_name="c", subcore_axis_name="s"),
        scratch_types=[
            pltpu.VMEM((NCH, CH), jnp.int32),
            pltpu.VMEM((CH, D_ACC), jnp.float32),
            pltpu.VMEM((CH, D_ACC), jnp.float32),
            pltpu.VMEM((CH, D_ACC), jnp.float32),
            pltpu.VMEM_SHARED((N_NODES_PAD, D_ACC), jnp.float32),
            pltpu.SemaphoreType.DMA,
            pltpu.SemaphoreType.DMA,
        ],
    )
    return kfn(h, row3, zeros_block)


def _combine_body(p_ref, w_ref, b_ref, o_ref):
    h = p_ref[0] + p_ref[1]
    o_ref[...] = jnp.dot(h, w_ref[...], preferred_element_type=jnp.float32)


def _combine(partials, w2, b2):
    blk = 1000
    return pl.pallas_call(
        _combine_body,
        grid=(N_NODES // blk,),
        in_specs=[
            pl.BlockSpec((NC, blk, D_ACC), lambda i: (0, i, 0)),
            pl.BlockSpec((D_HID, D_OUT), lambda i: (0, 0)),
            pl.BlockSpec((D_OUT,), lambda i: (0,)),
        ],
        out_specs=pl.BlockSpec((blk, D_OUT), lambda i: (i, 0)),
        out_shape=jax.ShapeDtypeStruct((N_NODES, D_OUT), jnp.float32),
    )(partials, w2, b2)


@jax.jit
def kernel(x, edge_index, edge_attr, W1, b1, W2, b2):
    row = edge_index[0].astype(jnp.int32)
    col = edge_index[1].astype(jnp.int32)
    row3 = row.reshape(NW, NCH, CH)
    col3 = col.reshape(NW, NCH, CH)
    w1x = W1[:D_FEAT, :]
    w1e = W1[D_FEAT:, :]
    zeros_block = jnp.zeros((ROWS_PER_SUB, D_ACC), jnp.float32)

    u = _node_proj(x, w1x, b1)
    g = _gather(u, col3)
    h = _edge_act(g, edge_attr.astype(jnp.bfloat16), w1e.astype(jnp.bfloat16))
    partials = _scatter(h, row3, zeros_block)
    return _combine(partials, W2, b2)


# speedup vs baseline: 2.0393x; 1.2929x over previous
import jax
import jax.numpy as jnp
from jax import lax
from jax.experimental import pallas as pl
from jax.experimental.pallas import tpu as pltpu
from jax.experimental.pallas import tpu_sc as plsc

N_NODES = 10000
N_EDGES = 320000
D_FEAT = 128
D_EDGE = 16
D_HID = 128
D_OUT = 128
D_ACC = D_HID

NC = 2
NS = 16
NW = NC * NS
PER_W = N_EDGES // NW
CH = 80
NCH = PER_W // CH
CH2 = 40
NCH2 = PER_W // CH2
NBUF = 4
Z_ROWS = 632
LAST_BASE = 15 * Z_ROWS
LAST_ROWS = N_NODES - LAST_BASE


def _node_proj_body(x_ref, w_ref, b_ref, u_ref):
    u_ref[...] = (
        jnp.dot(x_ref[...], w_ref[...], preferred_element_type=jnp.float32)
        + b_ref[...][None, :]
    )


def _node_proj(x, w1x, b1):
    blk = 1000
    return pl.pallas_call(
        _node_proj_body,
        grid=(N_NODES // blk,),
        in_specs=[
            pl.BlockSpec((blk, D_FEAT), lambda i: (i, 0)),
            pl.BlockSpec((D_FEAT, D_HID), lambda i: (0, 0)),
            pl.BlockSpec((D_HID,), lambda i: (0,)),
        ],
        out_specs=pl.BlockSpec((blk, D_HID), lambda i: (i, 0)),
        out_shape=jax.ShapeDtypeStruct((N_NODES, D_HID), jnp.float32),
    )(x, w1x, b1)


def _fused_body(u_hbm, t_hbm, col_hbm, row_hbm, z_hbm, p_hbm,
                cidx_v, rb0, rb1, rb2, t0, t1, t2, g0, g1, g2,
                acc_sh, seml, semg, semr, sems):
    c = lax.axis_index("c")
    s = lax.axis_index("s")
    wid = s * NC + c
    base = wid * PER_W

    @pl.when(s < NS - 1)
    def _():
        pltpu.sync_copy(z_hbm, acc_sh.at[pl.ds(s * Z_ROWS, Z_ROWS)])

    @pl.when(s == NS - 1)
    def _():
        pltpu.sync_copy(z_hbm.at[pl.ds(0, LAST_ROWS)],
                        acc_sh.at[pl.ds(LAST_BASE, LAST_ROWS)])

    pltpu.sync_copy(col_hbm.at[pl.ds(base, PER_W)], cidx_v)
    plsc.subcore_barrier()

    tbufs = (t0, t1, t2)
    gbufs = (g0, g1, g2)
    rbufs = (rb0, rb1, rb2)

    def start_pair(j, p):
        pltpu.async_copy(t_hbm.at[pl.ds(base + j * CH2, CH2)], tbufs[p], seml)
        pltpu.async_copy(row_hbm.at[pl.ds(base + j * CH2, CH2)], rbufs[p], semr)
        pltpu.async_copy(u_hbm.at[cidx_v.at[pl.ds(j * CH2, CH2)]], gbufs[p], semg)

    def wait_pair(p):
        pltpu.make_async_copy(t_hbm.at[pl.ds(base, CH2)], tbufs[p], seml).wait()
        pltpu.make_async_copy(row_hbm.at[pl.ds(base, CH2)], rbufs[p], semr).wait()
        pltpu.make_async_copy(u_hbm.at[pl.ds(0, CH2)], gbufs[p], semg).wait()

    def start_scat(j, p):
        pltpu.async_copy(tbufs[p], acc_sh.at[rbufs[p]], sems, add=True)

    def wait_scat(p):
        pltpu.make_async_copy(tbufs[p], acc_sh.at[pl.ds(0, CH2)], sems).wait()

    def relu_add(p):
        tb = tbufs[p]
        gb = gbufs[p]

        @pl.loop(0, CH2)
        def _(i):
            for cc in range(D_HID // 16):
                sl = (i, pl.ds(cc * 16, 16))
                tb[sl] = jnp.maximum(tb[sl] + gb[sl], 0.0)

    start_pair(0, 0)
    start_pair(1, 1)

    @pl.loop(0, 83)
    def _(k):
        j0 = k * 3
        for t in range(3):
            j = j0 + t
            p = t
            wait_pair(p)

            @pl.when(j >= 1)
            def _():
                wait_scat((t + 2) % 3)

            @pl.when(j + 2 < NCH2)
            def _():
                start_pair(j + 2, (t + 2) % 3)

            relu_add(p)
            start_scat(j, p)

    wait_pair(0)
    wait_scat(2)
    relu_add(0)
    start_scat(249, 0)
    wait_scat(0)

    plsc.subcore_barrier()

    @pl.when(s < NS - 1)
    def _():
        pltpu.sync_copy(acc_sh.at[pl.ds(s * Z_ROWS, Z_ROWS)],
                        p_hbm.at[c, pl.ds(s * Z_ROWS, Z_ROWS)])

    @pl.when(s == NS - 1)
    def _():
        pltpu.sync_copy(acc_sh.at[pl.ds(LAST_BASE, LAST_ROWS)],
                        p_hbm.at[c, pl.ds(LAST_BASE, LAST_ROWS)])


def _fused(u, t, col, row, zeros_block):
    kfn = pl.kernel(
        _fused_body,
        out_type=jax.ShapeDtypeStruct((NC, N_NODES, D_ACC), jnp.float32),
        mesh=plsc.VectorSubcoreMesh(core_axis_name="c", subcore_axis_name="s"),
        scratch_types=[
            pltpu.VMEM((PER_W,), jnp.int32),
            pltpu.VMEM((CH2,), jnp.int32),
            pltpu.VMEM((CH2,), jnp.int32),
            pltpu.VMEM((CH2,), jnp.int32),
            pltpu.VMEM((CH2, D_HID), jnp.float32),
            pltpu.VMEM((CH2, D_HID), jnp.float32),
            pltpu.VMEM((CH2, D_HID), jnp.float32),
            pltpu.VMEM((CH2, D_HID), jnp.float32),
            pltpu.VMEM((CH2, D_HID), jnp.float32),
            pltpu.VMEM((CH2, D_HID), jnp.float32),
            pltpu.VMEM_SHARED((N_NODES, D_ACC), jnp.float32),
            pltpu.SemaphoreType.DMA,
            pltpu.SemaphoreType.DMA,
            pltpu.SemaphoreType.DMA,
            pltpu.SemaphoreType.DMA,
        ],
    )
    return kfn(u, t, col, row, zeros_block)


def _edge_term_body(ea_ref, w_ref, t_ref):
    t_ref[...] = jnp.dot(ea_ref[...], w_ref[...], preferred_element_type=jnp.float32)


def _edge_term(ea, w1e):
    blk = 2000
    return pl.pallas_call(
        _edge_term_body,
        grid=(N_EDGES // blk,),
        in_specs=[
            pl.BlockSpec((blk, D_EDGE), lambda i: (i, 0)),
            pl.BlockSpec((D_EDGE, D_HID), lambda i: (0, 0)),
        ],
        out_specs=pl.BlockSpec((blk, D_HID), lambda i: (i, 0)),
        out_shape=jax.ShapeDtypeStruct((N_EDGES, D_HID), jnp.float32),
    )(ea, w1e)


def _combine_body(p_ref, w_ref, b_ref, o_ref):
    h = p_ref[0] + p_ref[1]
    o_ref[...] = jnp.dot(h, w_ref[...], preferred_element_type=jnp.float32)


def _combine(partials, w2, b2):
    blk = 1000
    return pl.pallas_call(
        _combine_body,
        grid=(N_NODES // blk,),
        in_specs=[
            pl.BlockSpec((NC, blk, D_ACC), lambda i: (0, i, 0)),
            pl.BlockSpec((D_HID, D_OUT), lambda i: (0, 0)),
            pl.BlockSpec((D_OUT,), lambda i: (0,)),
        ],
        out_specs=pl.BlockSpec((blk, D_OUT), lambda i: (i, 0)),
        out_shape=jax.ShapeDtypeStruct((N_NODES, D_OUT), jnp.float32),
    )(partials, w2, b2)


@jax.jit
def kernel(x, edge_index, edge_attr, W1, b1, W2, b2):
    row = edge_index[0].astype(jnp.int32)
    col = edge_index[1].astype(jnp.int32)
    w1x = W1[:D_FEAT, :]
    w1e = W1[D_FEAT:, :]
    zeros_block = jnp.zeros((Z_ROWS, D_ACC), jnp.float32)

    u = _node_proj(x, w1x, b1)
    t = _edge_term(edge_attr.astype(jnp.bfloat16), w1e.astype(jnp.bfloat16))
    partials = _fused(u, t, col, row, zeros_block)
    return _combine(partials, W2, b2)
